# baseline (speedup 1.0000x reference)
import jax
import jax.numpy as jnp
from jax.experimental import pallas as pl
from jax.experimental.pallas import tpu as pltpu

NUM_EXPERTS = 8
TOP_K = 2
HIDDEN = 2048
INTER = 16

BLOCK_T = 1024
N_SPLIT = 2
BLOCK_H = HIDDEN // N_SPLIT


def _moe_body(x_ref, rw_ref, dw_ref, up_ref, ex_ref, o_ref, h_ref):
    @pl.when(pl.program_id(1) == 0)
    def _compute_h():
        x = x_ref[...]
        lg_t = jax.lax.dot_general(
            rw_ref[...], x, (((1,), (1,)), ((), ())),
            preferred_element_type=jnp.float32,
        )
        mx = jnp.max(lg_t, axis=0, keepdims=True)
        e = jnp.exp(lg_t - mx)
        w = e / jnp.sum(e, axis=0, keepdims=True)
        row = jax.lax.broadcasted_iota(jnp.int32, w.shape, 0)
        m1 = jnp.max(w, axis=0, keepdims=True)
        idx1 = jnp.min(jnp.where(w >= m1, row, NUM_EXPERTS), axis=0, keepdims=True)
        sel1 = row == idx1
        w2 = jnp.where(sel1, -1.0, w)
        m2 = jnp.max(w2, axis=0, keepdims=True)
        idx2 = jnp.min(jnp.where(w2 >= m2, row, NUM_EXPERTS), axis=0, keepdims=True)
        sel2 = row == idx2
        s_t = jnp.where(sel1 | sel2, w, 0.0)
        s_exp = jax.lax.dot_general(
            s_t, ex_ref[...], (((0,), (0,)), ((), ())),
            preferred_element_type=jnp.float32,
        )
        h = jnp.dot(x, dw_ref[...], preferred_element_type=jnp.float32)
        h_ref[...] = h * jax.nn.sigmoid(h) * s_exp

    o_ref[...] = jnp.dot(h_ref[...], up_ref[...],
                         preferred_element_type=jnp.float32)


def kernel(x, router_w, down_w, up_w):
    t = x.shape[0]
    dw_t = down_w.reshape(NUM_EXPERTS * INTER, HIDDEN).T
    up_all = jnp.transpose(up_w, (0, 2, 1)).reshape(NUM_EXPERTS * INTER, HIDDEN)
    expand = (
        jax.lax.broadcasted_iota(jnp.int32, (NUM_EXPERTS, NUM_EXPERTS * INTER), 1)
        // INTER
        == jax.lax.broadcasted_iota(jnp.int32, (NUM_EXPERTS, NUM_EXPERTS * INTER), 0)
    ).astype(jnp.float32)

    grid = (t // BLOCK_T, N_SPLIT)
    return pl.pallas_call(
        _moe_body,
        grid=grid,
        in_specs=[
            pl.BlockSpec((BLOCK_T, HIDDEN), lambda i, j: (i, 0)),
            pl.BlockSpec((NUM_EXPERTS, HIDDEN), lambda i, j: (0, 0)),
            pl.BlockSpec((HIDDEN, NUM_EXPERTS * INTER), lambda i, j: (0, 0)),
            pl.BlockSpec((NUM_EXPERTS * INTER, BLOCK_H), lambda i, j: (0, j)),
            pl.BlockSpec((NUM_EXPERTS, NUM_EXPERTS * INTER), lambda i, j: (0, 0)),
        ],
        out_specs=pl.BlockSpec((BLOCK_T, BLOCK_H), lambda i, j: (i, j)),
        out_shape=jax.ShapeDtypeStruct((t, HIDDEN), x.dtype),
        scratch_shapes=[pltpu.VMEM((BLOCK_T, NUM_EXPERTS * INTER), jnp.float32)],
        compiler_params=pltpu.CompilerParams(
            dimension_semantics=("parallel", "arbitrary"),
        ),
    )(x, router_w, dw_t, up_all, expand)


# speedup vs baseline: 1.4244x; 1.4244x over previous
import jax
import jax.numpy as jnp
from jax.experimental import pallas as pl
from jax.experimental.pallas import tpu as pltpu

NUM_EXPERTS = 8
TOP_K = 2
HIDDEN = 2048
INTER = 16

BLOCK_T = 1024
HALF_H = HIDDEN // 2


def _moe_body(xa_ref, xb_ref, rw_ref, dw_ref, up_ref, ex_ref, o_ref):
    xa = xa_ref[...]
    xb = xb_ref[...]
    lg_t = jax.lax.dot_general(
        rw_ref[:, :HALF_H], xa, (((1,), (1,)), ((), ())),
        preferred_element_type=jnp.float32,
    ) + jax.lax.dot_general(
        rw_ref[:, HALF_H:], xb, (((1,), (1,)), ((), ())),
        preferred_element_type=jnp.float32,
    )
    mx = jnp.max(lg_t, axis=0, keepdims=True)
    e = jnp.exp(lg_t - mx)
    w = e / jnp.sum(e, axis=0, keepdims=True)
    row = jax.lax.broadcasted_iota(jnp.int32, w.shape, 0)
    m1 = jnp.max(w, axis=0, keepdims=True)
    idx1 = jnp.min(jnp.where(w >= m1, row, NUM_EXPERTS), axis=0, keepdims=True)
    sel1 = row == idx1
    w2 = jnp.where(sel1, -1.0, w)
    m2 = jnp.max(w2, axis=0, keepdims=True)
    idx2 = jnp.min(jnp.where(w2 >= m2, row, NUM_EXPERTS), axis=0, keepdims=True)
    sel2 = row == idx2
    s_t = jnp.where(sel1 | sel2, w, 0.0)
    s_exp = jax.lax.dot_general(
        s_t, ex_ref[...], (((0,), (0,)), ((), ())),
        preferred_element_type=jnp.float32,
    )
    h = (
        jnp.dot(xa, dw_ref[:HALF_H, :], preferred_element_type=jnp.float32)
        + jnp.dot(xb, dw_ref[HALF_H:, :], preferred_element_type=jnp.float32)
    )
    h = h * jax.nn.sigmoid(h) * s_exp
    o_ref[...] = jnp.dot(h, up_ref[...], preferred_element_type=jnp.float32)


def kernel(x, router_w, down_w, up_w):
    t = x.shape[0]
    dw_t = down_w.reshape(NUM_EXPERTS * INTER, HIDDEN).T
    up_all = jnp.transpose(up_w, (0, 2, 1)).reshape(NUM_EXPERTS * INTER, HIDDEN)
    expand = (
        jax.lax.broadcasted_iota(jnp.int32, (NUM_EXPERTS, NUM_EXPERTS * INTER), 1)
        // INTER
        == jax.lax.broadcasted_iota(jnp.int32, (NUM_EXPERTS, NUM_EXPERTS * INTER), 0)
    ).astype(jnp.float32)

    grid = (t // BLOCK_T,)
    return pl.pallas_call(
        _moe_body,
        grid=grid,
        in_specs=[
            pl.BlockSpec((BLOCK_T, HALF_H), lambda i: (i, 0)),
            pl.BlockSpec((BLOCK_T, HALF_H), lambda i: (i, 1)),
            pl.BlockSpec((NUM_EXPERTS, HIDDEN), lambda i: (0, 0)),
            pl.BlockSpec((HIDDEN, NUM_EXPERTS * INTER), lambda i: (0, 0)),
            pl.BlockSpec((NUM_EXPERTS * INTER, HIDDEN), lambda i: (0, 0)),
            pl.BlockSpec((NUM_EXPERTS, NUM_EXPERTS * INTER), lambda i: (0, 0)),
        ],
        out_specs=pl.BlockSpec((BLOCK_T, HIDDEN), lambda i: (i, 0)),
        out_shape=jax.ShapeDtypeStruct((t, HIDDEN), x.dtype),
        compiler_params=pltpu.CompilerParams(
            dimension_semantics=("parallel",),
        ),
    )(x, x, router_w, dw_t, up_all, expand)


# speedup vs baseline: 1.4844x; 1.0421x over previous
import jax
import jax.numpy as jnp
from jax.experimental import pallas as pl
from jax.experimental.pallas import tpu as pltpu

NUM_EXPERTS = 8
TOP_K = 2
HIDDEN = 2048
INTER = 16

BLOCK_T = 1024


def _moe_body(x_ref, dwa_ref, up_ref, ex_ref, o_ref):
    x = x_ref[...]
    ha = jnp.dot(x, dwa_ref[...], preferred_element_type=jnp.float32)
    h = ha[:, : NUM_EXPERTS * INTER]
    lg = ha[:, NUM_EXPERTS * INTER :]
    lg_t = lg.T
    mx = jnp.max(lg_t, axis=0, keepdims=True)
    e = jnp.exp(lg_t - mx)
    w = e / jnp.sum(e, axis=0, keepdims=True)
    row = jax.lax.broadcasted_iota(jnp.int32, w.shape, 0)
    m1 = jnp.max(w, axis=0, keepdims=True)
    idx1 = jnp.min(jnp.where(w >= m1, row, NUM_EXPERTS), axis=0, keepdims=True)
    sel1 = row == idx1
    w2 = jnp.where(sel1, -1.0, w)
    m2 = jnp.max(w2, axis=0, keepdims=True)
    idx2 = jnp.min(jnp.where(w2 >= m2, row, NUM_EXPERTS), axis=0, keepdims=True)
    sel2 = row == idx2
    s_t = jnp.where(sel1 | sel2, w, 0.0)
    s_exp = jax.lax.dot_general(
        s_t, ex_ref[...], (((0,), (0,)), ((), ())),
        preferred_element_type=jnp.float32,
    )
    hs = h * jax.nn.sigmoid(h) * s_exp
    o_ref[...] = jnp.dot(hs, up_ref[...], preferred_element_type=jnp.float32)


def kernel(x, router_w, down_w, up_w):
    t = x.shape[0]
    dw_t = down_w.reshape(NUM_EXPERTS * INTER, HIDDEN).T
    dwa = jnp.concatenate([dw_t, router_w.T], axis=1)
    up_all = jnp.transpose(up_w, (0, 2, 1)).reshape(NUM_EXPERTS * INTER, HIDDEN)
    expand = (
        jax.lax.broadcasted_iota(jnp.int32, (NUM_EXPERTS, NUM_EXPERTS * INTER), 1)
        // INTER
        == jax.lax.broadcasted_iota(jnp.int32, (NUM_EXPERTS, NUM_EXPERTS * INTER), 0)
    ).astype(jnp.float32)

    grid = (t // BLOCK_T,)
    return pl.pallas_call(
        _moe_body,
        grid=grid,
        in_specs=[
            pl.BlockSpec((BLOCK_T, HIDDEN), lambda i: (i, 0)),
            pl.BlockSpec((HIDDEN, NUM_EXPERTS * INTER + NUM_EXPERTS), lambda i: (0, 0)),
            pl.BlockSpec((NUM_EXPERTS * INTER, HIDDEN), lambda i: (0, 0)),
            pl.BlockSpec((NUM_EXPERTS, NUM_EXPERTS * INTER), lambda i: (0, 0)),
        ],
        out_specs=pl.BlockSpec((BLOCK_T, HIDDEN), lambda i: (i, 0)),
        out_shape=jax.ShapeDtypeStruct((t, HIDDEN), x.dtype),
        compiler_params=pltpu.CompilerParams(
            dimension_semantics=("parallel",),
        ),
    )(x, dwa, up_all, expand)
